# baseline (speedup 1.0000x reference)
import jax
import jax.numpy as jnp
from jax import lax
from jax.experimental import pallas as pl
from jax.experimental.pallas import tpu as pltpu
from jax.experimental.pallas import tpu_sc as plsc

MAXLEN = 8192
OUTPUT_DIM = 2048

_NC = 2
_ROWS_PER_SC = MAXLEN // _NC
_CHUNK = 128
_NCHUNKS = _ROWS_PER_SC // _CHUNK
_NBUF = 4


def _sc_copy(table_hbm, out_hbm, spbuf, in_s0, in_s1, in_s2, in_s3,
             out_s0, out_s1, out_s2, out_s3):
    cid = lax.axis_index("c")
    base = cid * _ROWS_PER_SC
    in_sems = (in_s0, in_s1, in_s2, in_s3)
    out_sems = (out_s0, out_s1, out_s2, out_s3)

    def cin(i):
        return pltpu.make_async_copy(
            table_hbm.at[pl.ds(base + i * _CHUNK, _CHUNK)],
            spbuf.at[i % _NBUF], in_sems[i % _NBUF])

    def cout(i):
        return pltpu.make_async_copy(
            spbuf.at[i % _NBUF],
            out_hbm.at[pl.ds(base + i * _CHUNK, _CHUNK)],
            out_sems[i % _NBUF])

    for i in range(_NBUF):
        cin(i).start()
    for i in range(_NCHUNKS):
        cin(i).wait()
        cout(i).start()
        if i + _NBUF < _NCHUNKS:
            cout(i).wait()
            cin(i + _NBUF).start()
    for i in range(_NCHUNKS - _NBUF, _NCHUNKS):
        cout(i).wait()


def kernel(inputs, table):
    del inputs
    mesh = plsc.ScalarSubcoreMesh(axis_name="c", num_cores=_NC)
    out = pl.kernel(
        _sc_copy,
        mesh=mesh,
        out_type=jax.ShapeDtypeStruct((MAXLEN, OUTPUT_DIM), table.dtype),
        scratch_types=[
            pltpu.MemorySpace.VMEM_SHARED((_NBUF, _CHUNK, OUTPUT_DIM),
                                          jnp.float32),
            pltpu.SemaphoreType.DMA,
            pltpu.SemaphoreType.DMA,
            pltpu.SemaphoreType.DMA,
            pltpu.SemaphoreType.DMA,
            pltpu.SemaphoreType.DMA,
            pltpu.SemaphoreType.DMA,
            pltpu.SemaphoreType.DMA,
            pltpu.SemaphoreType.DMA,
        ],
    )(table)
    return out[None]


# speedup vs baseline: 1.1526x; 1.1526x over previous
import jax
import jax.numpy as jnp
from jax import lax
from jax.experimental import pallas as pl
from jax.experimental.pallas import tpu as pltpu
from jax.experimental.pallas import tpu_sc as plsc

MAXLEN = 8192
OUTPUT_DIM = 2048

_NC = 2
_ROWS_PER_SC = MAXLEN // _NC
_NISS = 8
_ROWS_PER_ISS = _ROWS_PER_SC // _NISS
_CHUNK = 32
_NCHUNKS = _ROWS_PER_ISS // _CHUNK
_NBUF = 2


def _sc_copy(table_hbm, out_hbm, spbuf, in_s0, in_s1, out_s0, out_s1):
    cid = lax.axis_index("c")
    sid = lax.axis_index("s")
    base = cid * _ROWS_PER_SC + sid * _ROWS_PER_ISS
    in_sems = (in_s0, in_s1)
    out_sems = (out_s0, out_s1)

    def cin(i):
        return pltpu.make_async_copy(
            table_hbm.at[pl.ds(base + i * _CHUNK, _CHUNK)],
            spbuf.at[sid, i % _NBUF], in_sems[i % _NBUF])

    def cout(i):
        return pltpu.make_async_copy(
            spbuf.at[sid, i % _NBUF],
            out_hbm.at[pl.ds(base + i * _CHUNK, _CHUNK)],
            out_sems[i % _NBUF])

    @pl.when(sid < _NISS)
    def _():
        cin(0).start()
        for i in range(_NCHUNKS):
            if i + 1 < _NCHUNKS:
                if i >= 1:
                    cout(i - 1).wait()
                cin(i + 1).start()
            cin(i).wait()
            cout(i).start()
        cout(_NCHUNKS - 2).wait()
        cout(_NCHUNKS - 1).wait()


def kernel(inputs, table):
    del inputs
    mesh = plsc.VectorSubcoreMesh(core_axis_name="c", subcore_axis_name="s")
    out = pl.kernel(
        _sc_copy,
        mesh=mesh,
        out_type=jax.ShapeDtypeStruct((MAXLEN, OUTPUT_DIM), table.dtype),
        scratch_types=[
            pltpu.MemorySpace.VMEM_SHARED((_NISS, _NBUF, _CHUNK, OUTPUT_DIM),
                                          jnp.float32),
            pltpu.SemaphoreType.DMA,
            pltpu.SemaphoreType.DMA,
            pltpu.SemaphoreType.DMA,
            pltpu.SemaphoreType.DMA,
        ],
    )(table)
    return out[None]


# speedup vs baseline: 1.1558x; 1.0028x over previous
import jax
import jax.numpy as jnp
from jax import lax
from jax.experimental import pallas as pl
from jax.experimental.pallas import tpu as pltpu
from jax.experimental.pallas import tpu_sc as plsc

MAXLEN = 8192
OUTPUT_DIM = 2048

_NC = 2
_ROWS_PER_SC = MAXLEN // _NC
_NISS = 16
_ROWS_PER_ISS = _ROWS_PER_SC // _NISS
_CHUNK = 16
_NCHUNKS = _ROWS_PER_ISS // _CHUNK
_NBUF = 2


def _sc_copy(table_hbm, out_hbm, spbuf, in_s0, in_s1, out_s0, out_s1):
    cid = lax.axis_index("c")
    sid = lax.axis_index("s")
    base = cid * _ROWS_PER_SC + sid * _ROWS_PER_ISS
    in_sems = (in_s0, in_s1)
    out_sems = (out_s0, out_s1)

    def cin(i):
        return pltpu.make_async_copy(
            table_hbm.at[pl.ds(base + i * _CHUNK, _CHUNK)],
            spbuf.at[sid, i % _NBUF], in_sems[i % _NBUF])

    def cout(i):
        return pltpu.make_async_copy(
            spbuf.at[sid, i % _NBUF],
            out_hbm.at[pl.ds(base + i * _CHUNK, _CHUNK)],
            out_sems[i % _NBUF])

    @pl.when(sid < _NISS)
    def _():
        cin(0).start()
        for i in range(_NCHUNKS):
            if i + 1 < _NCHUNKS:
                if i >= 1:
                    cout(i - 1).wait()
                cin(i + 1).start()
            cin(i).wait()
            cout(i).start()
        cout(_NCHUNKS - 2).wait()
        cout(_NCHUNKS - 1).wait()


def kernel(inputs, table):
    del inputs
    mesh = plsc.VectorSubcoreMesh(core_axis_name="c", subcore_axis_name="s")
    out = pl.kernel(
        _sc_copy,
        mesh=mesh,
        out_type=jax.ShapeDtypeStruct((MAXLEN, OUTPUT_DIM), table.dtype),
        scratch_types=[
            pltpu.MemorySpace.VMEM_SHARED((_NISS, _NBUF, _CHUNK, OUTPUT_DIM),
                                          jnp.float32),
            pltpu.SemaphoreType.DMA,
            pltpu.SemaphoreType.DMA,
            pltpu.SemaphoreType.DMA,
            pltpu.SemaphoreType.DMA,
        ],
    )(table)
    return out[None]
